# baseline (speedup 1.0000x reference)
import functools

import jax
import jax.numpy as jnp
from jax import lax
from jax.experimental import pallas as pl
from jax.experimental.pallas import tpu as pltpu
from jax.experimental.pallas import tpu_sc as plsc

_NC, _NS = 2, 16
_NW = _NC * _NS
_SUB = 128
_NSUB = 4
_CHUNK = _SUB * _NSUB


def _sc_gather(table, idx, r_pad):
    bpw = r_pad // _NW
    nchunk = bpw // _CHUNK
    mesh = plsc.VectorSubcoreMesh(core_axis_name="c", subcore_axis_name="s")

    @functools.partial(
        pl.kernel,
        mesh=mesh,
        out_type=jax.ShapeDtypeStruct((r_pad, 128), jnp.float32),
        scratch_types=[
            pltpu.VMEM((_CHUNK,), jnp.int32),
            pltpu.VMEM((_CHUNK, 128), jnp.float32),
            pltpu.SemaphoreType.DMA,
        ],
    )
    def gk(table_hbm, idx_hbm, out_hbm, idx_v, rows_v, sem):
        wid = lax.axis_index("s") * _NC + lax.axis_index("c")
        base = wid * bpw

        def body(t, carry):
            off = pl.multiple_of(base + t * _CHUNK, _CHUNK)
            pltpu.sync_copy(idx_hbm.at[pl.ds(off, _CHUNK)], idx_v)
            descs = [
                pltpu.async_copy(table_hbm.at[idx_v.at[pl.ds(i * _SUB, _SUB)]],
                                 rows_v.at[pl.ds(i * _SUB, _SUB)], sem)
                for i in range(_NSUB)
            ]
            for d_ in descs:
                d_.wait()
            pltpu.sync_copy(rows_v, out_hbm.at[pl.ds(off, _CHUNK)])
            return carry

        lax.fori_loop(0, nchunk, body, 0)

    return gk(table, idx)


_PAIRS = [(0, 1), (0, 2), (0, 3), (1, 2), (1, 3), (2, 3)]
_TRIPLES = [(i, j, k) for i in range(3) for j in range(i + 1, 4)
            for k in range(j + 1, 5)]


def _tc_body(g_ref, w1t_ref, w2t_ref, wot_ref, attv_ref,
             out_ref, score_ref):
    _, kp, t, c = g_ref.shape
    gi = g_ref[0]
    gj = g_ref[1]
    w1t = w1t_ref[...]
    w2t = w2t_ref[...]
    wot = wot_ref[...]
    attv = attv_ref[...]

    def mm(a3, w):
        a2 = a3.reshape(kp * t, c).astype(jnp.bfloat16)
        r = lax.dot_general(a2, w.astype(jnp.bfloat16), (((1,), (0,)), ((), ())),
                            preferred_element_type=jnp.float32,
                            precision=lax.Precision.DEFAULT)
        return r.reshape(kp, t, c)

    a1_0 = mm(gi, w1t)
    a1_1 = mm(gj, w1t)
    a2_0 = mm(gi, w2t)
    a2_1 = mm(gj, w2t)
    a1s = [a1_0, a1_1, a1_0 - a1_1]
    d = gi - gj
    nr = jnp.sqrt(jnp.sum(d * d, axis=2, keepdims=True))
    w2sum = jnp.sum(w2t, axis=0, keepdims=True)[None]
    a2s = [a2_0, a2_1, a2_0 - a2_1,
           a2_0 - jnp.mean(a2_0, axis=0, keepdims=True),
           nr * w2sum]

    firs = {}
    f1s = {}
    for (i, j) in _PAIRS:
        fir = jnp.maximum(a1s[i] + a2s[j], 0.0)
        firs[(i, j)] = fir
        f1s[(i, j)] = mm(fir, w1t)
    nn = firs[(0, 1)]

    nn_hi = jnp.max(nn, axis=0, keepdims=True)
    nn_lo = jnp.min(nn, axis=0, keepdims=True)
    contribs = []
    for o, (i, j, k) in enumerate(_TRIPLES):
        op = jnp.maximum(f1s[(i, j)] + a2s[k], 0.0)
        out_ref[o] = jnp.maximum(jnp.max(mm(op, wot), axis=0), 0.0)
        cv = attv[o][None, None, :]
        base = jnp.where(cv >= 0, nn_hi, nn_lo)
        e = jnp.exp(cv * (nn - base))
        s = jnp.sum(e, axis=0)
        w_ = jnp.sum(e * op, axis=0)
        contribs.append(jnp.sum(w_ / s, axis=0)[None, :])
    contribs = jnp.concatenate(
        contribs + [jnp.zeros((6, c), jnp.float32)], axis=0)

    @pl.when(pl.program_id(0) == 0)
    def _():
        score_ref[...] = contribs

    @pl.when(pl.program_id(0) != 0)
    def _():
        score_ref[...] = score_ref[...] + contribs


def kernel(x, edge_index, W_nn, W_out, att):
    n = x.shape[2]
    c = x.shape[1]
    k = edge_index.shape[3]
    kp = k + 1
    xT = jnp.transpose(x[0, :, :, 0])
    ei = edge_index.astype(jnp.int32)
    self_row = jnp.arange(n, dtype=jnp.int32)[None]
    idx_i = jnp.concatenate([jnp.transpose(ei[1, 0]), self_row], axis=0)
    idx_j = jnp.concatenate([jnp.transpose(ei[0, 0]), self_row], axis=0)
    r = 2 * kp * n
    r_pad = ((r + _NW * _CHUNK - 1) // (_NW * _CHUNK)) * (_NW * _CHUNK)
    idx_all = jnp.concatenate(
        [idx_i.reshape(-1), idx_j.reshape(-1),
         jnp.zeros((r_pad - r,), jnp.int32)])

    g = _sc_gather(xT, idx_all, r_pad)
    g4 = g[:r].reshape(2, kp, n, c)

    w1t = jnp.transpose(W_nn[:, :c])
    w2t = jnp.transpose(W_nn[:, c:])
    wot = jnp.transpose(W_out)
    cv = jnp.where(att >= 0, att, 0.2 * att).reshape(10, 1)
    attv = jnp.zeros((16, c), jnp.float32).at[:10].set(
        jnp.broadcast_to(cv, (10, c)))

    t_ = 200
    out_all, score_part = pl.pallas_call(
        _tc_body,
        grid=(n // t_,),
        in_specs=[
            pl.BlockSpec((2, kp, t_, c), lambda i: (0, 0, i, 0)),
            pl.BlockSpec((c, c), lambda i: (0, 0)),
            pl.BlockSpec((c, c), lambda i: (0, 0)),
            pl.BlockSpec((c, c), lambda i: (0, 0)),
            pl.BlockSpec((16, c), lambda i: (0, 0)),
        ],
        out_specs=[
            pl.BlockSpec((10, t_, c), lambda i: (0, i, 0)),
            pl.BlockSpec((16, c), lambda i: (0, 0)),
        ],
        out_shape=[
            jax.ShapeDtypeStruct((10, n, c), jnp.float32),
            jax.ShapeDtypeStruct((16, c), jnp.float32),
        ],
    )(g4, w1t, w2t, wot, attv)

    score = jnp.sum(score_part[:10], axis=1)
    index = jnp.argmax(score)
    sel = jnp.take(out_all, index, axis=0)
    return jnp.transpose(sel)[None, :, :, None]


# speedup vs baseline: 1.0267x; 1.0267x over previous
import functools

import jax
import jax.numpy as jnp
from jax import lax
from jax.experimental import pallas as pl
from jax.experimental.pallas import tpu as pltpu
from jax.experimental.pallas import tpu_sc as plsc

_NC, _NS = 2, 16
_NW = _NC * _NS
_SUB = 128
_NSUB = 3
_CHUNK = _SUB * _NSUB


def _sc_gather(table, idx, r_pad):
    w = table.shape[1]
    bpw = r_pad // _NW
    nchunk = bpw // _CHUNK
    mesh = plsc.VectorSubcoreMesh(core_axis_name="c", subcore_axis_name="s")

    @functools.partial(
        pl.kernel,
        mesh=mesh,
        out_type=jax.ShapeDtypeStruct((r_pad, w), jnp.float32),
        scratch_types=[
            pltpu.VMEM((_CHUNK,), jnp.int32),
            pltpu.VMEM((_CHUNK,), jnp.int32),
            pltpu.VMEM((_CHUNK, w), jnp.float32),
            pltpu.VMEM((_CHUNK, w), jnp.float32),
            pltpu.SemaphoreType.DMA,
            pltpu.SemaphoreType.DMA,
            pltpu.SemaphoreType.DMA,
            pltpu.SemaphoreType.DMA,
        ],
    )
    def gk(table_hbm, idx_hbm, out_hbm, idx0, idx1, rows0, rows1,
           sg0, sg1, sw0, sw1):
        wid = lax.axis_index("s") * _NC + lax.axis_index("c")
        base = wid * bpw
        npair = nchunk // 2

        def fire_gather(idx_v, rows_v, sem):
            for i in range(_NSUB):
                pltpu.async_copy(
                    table_hbm.at[idx_v.at[pl.ds(i * _SUB, _SUB)]],
                    rows_v.at[pl.ds(i * _SUB, _SUB)], sem)

        def wait_gather(idx_v, rows_v, sem):
            for i in range(_NSUB):
                pltpu.make_async_copy(
                    table_hbm.at[idx_v.at[pl.ds(i * _SUB, _SUB)]],
                    rows_v.at[pl.ds(i * _SUB, _SUB)], sem).wait()

        pltpu.sync_copy(idx_hbm.at[pl.ds(base, _CHUNK)], idx0)
        fire_gather(idx0, rows0, sg0)

        def body(g, carry):
            off0 = pl.multiple_of(base + (2 * g) * _CHUNK, _CHUNK)
            off1 = pl.multiple_of(base + (2 * g + 1) * _CHUNK, _CHUNK)
            off2 = pl.multiple_of(base + (2 * g + 2) * _CHUNK, _CHUNK)
            pltpu.sync_copy(idx_hbm.at[pl.ds(off1, _CHUNK)], idx1)

            @pl.when(g > 0)
            def _():
                pltpu.make_async_copy(
                    rows1, out_hbm.at[pl.ds(off1 - 2 * _CHUNK, _CHUNK)],
                    sw1).wait()

            wait_gather(idx0, rows0, sg0)
            pltpu.async_copy(rows0, out_hbm.at[pl.ds(off0, _CHUNK)], sw0)
            fire_gather(idx1, rows1, sg1)

            @pl.when(g < npair - 1)
            def _():
                pltpu.sync_copy(idx_hbm.at[pl.ds(off2, _CHUNK)], idx0)
                pltpu.make_async_copy(
                    rows0, out_hbm.at[pl.ds(off0, _CHUNK)], sw0).wait()

            wait_gather(idx1, rows1, sg1)
            pltpu.async_copy(rows1, out_hbm.at[pl.ds(off1, _CHUNK)], sw1)

            @pl.when(g < npair - 1)
            def _():
                fire_gather(idx0, rows0, sg0)

            return carry

        lax.fori_loop(0, npair, body, 0)
        last0 = pl.multiple_of(base + (nchunk - 2) * _CHUNK, _CHUNK)
        last1 = pl.multiple_of(base + (nchunk - 1) * _CHUNK, _CHUNK)
        pltpu.make_async_copy(rows0, out_hbm.at[pl.ds(last0, _CHUNK)], sw0).wait()
        pltpu.make_async_copy(rows1, out_hbm.at[pl.ds(last1, _CHUNK)], sw1).wait()

    return gk(table, idx)


_PAIRS = [(0, 1), (0, 2), (0, 3), (1, 2), (1, 3), (2, 3)]
_TRIPLES = [(i, j, k) for i in range(3) for j in range(i + 1, 4)
            for k in range(j + 1, 5)]


def _tc_body(g_ref, w1t_ref, w2t_ref, wot_ref, attv_ref,
             out_ref, score_ref):
    _, kp, t, c = g_ref.shape
    gi = g_ref[0]
    gj = g_ref[1]
    w1t = w1t_ref[...]
    w2t = w2t_ref[...]
    wot = wot_ref[...]
    attv = attv_ref[...]

    def mm(a3, w):
        a2 = a3.reshape(kp * t, c)
        if a2.dtype != jnp.bfloat16:
            a2 = a2.astype(jnp.bfloat16)
        r = lax.dot_general(a2, w.astype(jnp.bfloat16), (((1,), (0,)), ((), ())),
                            preferred_element_type=jnp.float32,
                            precision=lax.Precision.DEFAULT)
        return r.reshape(kp, t, c)

    a1_0 = mm(gi, w1t)
    a1_1 = mm(gj, w1t)
    a2_0 = mm(gi, w2t)
    a2_1 = mm(gj, w2t)
    a1s = [a1_0, a1_1, a1_0 - a1_1]
    d = gi - gj
    nr = jnp.sqrt(jnp.sum(d * d, axis=2, keepdims=True))
    w2sum = jnp.sum(w2t, axis=0, keepdims=True)[None]
    a2s = [a2_0, a2_1, a2_0 - a2_1,
           a2_0 - jnp.mean(a2_0, axis=0, keepdims=True),
           nr * w2sum]

    firs = {}
    f1s = {}
    for (i, j) in _PAIRS:
        fir = jnp.maximum(a1s[i] + a2s[j], 0.0)
        firs[(i, j)] = fir
        f1s[(i, j)] = mm(fir, w1t)
    nn = firs[(0, 1)]

    nn_hi = jnp.max(nn, axis=0, keepdims=True)
    nn_lo = jnp.min(nn, axis=0, keepdims=True)
    contribs = []
    for o, (i, j, k) in enumerate(_TRIPLES):
        op = jnp.maximum(f1s[(i, j)] + a2s[k], 0.0)
        out_ref[o] = jnp.maximum(jnp.max(mm(op, wot), axis=0), 0.0)
        cv = attv[o][None, None, :]
        base = jnp.where(cv >= 0, nn_hi, nn_lo)
        e = jnp.exp(cv * (nn - base))
        s = jnp.sum(e, axis=0)
        w_ = jnp.sum(e * op, axis=0)
        contribs.append(jnp.sum(w_ / s, axis=0)[None, :])
    contribs = jnp.concatenate(
        contribs + [jnp.zeros((6, c), jnp.float32)], axis=0)

    @pl.when(pl.program_id(0) == 0)
    def _():
        score_ref[...] = contribs

    @pl.when(pl.program_id(0) != 0)
    def _():
        score_ref[...] = score_ref[...] + contribs


def kernel(x, edge_index, W_nn, W_out, att):
    n = x.shape[2]
    c = x.shape[1]
    k = edge_index.shape[3]
    kp = k + 1
    xT = jnp.transpose(x[0, :, :, 0])
    ei = edge_index.astype(jnp.int32)
    self_row = jnp.arange(n, dtype=jnp.int32)[None]
    idx_i = jnp.concatenate([jnp.transpose(ei[1, 0]), self_row], axis=0)
    idx_j = jnp.concatenate([jnp.transpose(ei[0, 0]), self_row], axis=0)
    r = 2 * kp * n
    r_pad = ((r + _NW * _CHUNK - 1) // (_NW * _CHUNK)) * (_NW * _CHUNK)
    idx_all = jnp.concatenate(
        [idx_i.reshape(-1), idx_j.reshape(-1),
         jnp.zeros((r_pad - r,), jnp.int32)])

    g = _sc_gather(xT, idx_all, r_pad)
    g4 = g[:r].reshape(2, kp, n, c)

    w1t = jnp.transpose(W_nn[:, :c])
    w2t = jnp.transpose(W_nn[:, c:])
    wot = jnp.transpose(W_out)
    cv = jnp.where(att >= 0, att, 0.2 * att).reshape(10, 1)
    attv = jnp.zeros((16, c), jnp.float32).at[:10].set(
        jnp.broadcast_to(cv, (10, c)))

    t_ = 200
    out_all, score_part = pl.pallas_call(
        _tc_body,
        grid=(n // t_,),
        in_specs=[
            pl.BlockSpec((2, kp, t_, c), lambda i: (0, 0, i, 0)),
            pl.BlockSpec((c, c), lambda i: (0, 0)),
            pl.BlockSpec((c, c), lambda i: (0, 0)),
            pl.BlockSpec((c, c), lambda i: (0, 0)),
            pl.BlockSpec((16, c), lambda i: (0, 0)),
        ],
        out_specs=[
            pl.BlockSpec((10, t_, c), lambda i: (0, i, 0)),
            pl.BlockSpec((16, c), lambda i: (0, 0)),
        ],
        out_shape=[
            jax.ShapeDtypeStruct((10, n, c), jnp.float32),
            jax.ShapeDtypeStruct((16, c), jnp.float32),
        ],
    )(g4, w1t, w2t, wot, attv)

    score = jnp.sum(score_part[:10], axis=1)
    index = jnp.argmax(score)
    sel = jnp.take(out_all, index, axis=0)
    return jnp.transpose(sel)[None, :, :, None]


# speedup vs baseline: 1.1147x; 1.0857x over previous
import functools

import jax
import jax.numpy as jnp
from jax import lax
from jax.experimental import pallas as pl
from jax.experimental.pallas import tpu as pltpu
from jax.experimental.pallas import tpu_sc as plsc

_NC, _NS = 2, 16
_NW = _NC * _NS
_SUB = 128
_NSUB = 3
_CHUNK = _SUB * _NSUB


def _sc_gather(table, idx, r_pad):
    w = table.shape[1]
    bpw = r_pad // _NW
    nchunk = bpw // _CHUNK
    mesh = plsc.VectorSubcoreMesh(core_axis_name="c", subcore_axis_name="s")

    @functools.partial(
        pl.kernel,
        mesh=mesh,
        out_type=jax.ShapeDtypeStruct((r_pad, w), jnp.float32),
        scratch_types=[
            pltpu.VMEM((_CHUNK,), jnp.int32),
            pltpu.VMEM((_CHUNK,), jnp.int32),
            pltpu.VMEM((_CHUNK, w), jnp.float32),
            pltpu.VMEM((_CHUNK, w), jnp.float32),
            pltpu.SemaphoreType.DMA,
            pltpu.SemaphoreType.DMA,
            pltpu.SemaphoreType.DMA,
            pltpu.SemaphoreType.DMA,
        ],
    )
    def gk(table_hbm, idx_hbm, out_hbm, idx0, idx1, rows0, rows1,
           sg0, sg1, sw0, sw1):
        wid = lax.axis_index("s") * _NC + lax.axis_index("c")
        base = wid * bpw
        npair = nchunk // 2

        def fire_gather(idx_v, rows_v, sem):
            for i in range(_NSUB):
                pltpu.async_copy(
                    table_hbm.at[idx_v.at[pl.ds(i * _SUB, _SUB)]],
                    rows_v.at[pl.ds(i * _SUB, _SUB)], sem)

        def wait_gather(idx_v, rows_v, sem):
            for i in range(_NSUB):
                pltpu.make_async_copy(
                    table_hbm.at[idx_v.at[pl.ds(i * _SUB, _SUB)]],
                    rows_v.at[pl.ds(i * _SUB, _SUB)], sem).wait()

        pltpu.sync_copy(idx_hbm.at[pl.ds(base, _CHUNK)], idx0)
        fire_gather(idx0, rows0, sg0)

        def body(g, carry):
            off0 = pl.multiple_of(base + (2 * g) * _CHUNK, _CHUNK)
            off1 = pl.multiple_of(base + (2 * g + 1) * _CHUNK, _CHUNK)
            off2 = pl.multiple_of(base + (2 * g + 2) * _CHUNK, _CHUNK)
            pltpu.sync_copy(idx_hbm.at[pl.ds(off1, _CHUNK)], idx1)

            @pl.when(g > 0)
            def _():
                pltpu.make_async_copy(
                    rows1, out_hbm.at[pl.ds(off1 - 2 * _CHUNK, _CHUNK)],
                    sw1).wait()

            wait_gather(idx0, rows0, sg0)
            pltpu.async_copy(rows0, out_hbm.at[pl.ds(off0, _CHUNK)], sw0)
            fire_gather(idx1, rows1, sg1)

            @pl.when(g < npair - 1)
            def _():
                pltpu.sync_copy(idx_hbm.at[pl.ds(off2, _CHUNK)], idx0)
                pltpu.make_async_copy(
                    rows0, out_hbm.at[pl.ds(off0, _CHUNK)], sw0).wait()

            wait_gather(idx1, rows1, sg1)
            pltpu.async_copy(rows1, out_hbm.at[pl.ds(off1, _CHUNK)], sw1)

            @pl.when(g < npair - 1)
            def _():
                fire_gather(idx0, rows0, sg0)

            return carry

        lax.fori_loop(0, npair, body, 0)
        last0 = pl.multiple_of(base + (nchunk - 2) * _CHUNK, _CHUNK)
        last1 = pl.multiple_of(base + (nchunk - 1) * _CHUNK, _CHUNK)
        pltpu.make_async_copy(rows0, out_hbm.at[pl.ds(last0, _CHUNK)], sw0).wait()
        pltpu.make_async_copy(rows1, out_hbm.at[pl.ds(last1, _CHUNK)], sw1).wait()

    return gk(table, idx)


_PAIRS = [(0, 1), (0, 2), (0, 3), (1, 2), (1, 3), (2, 3)]
_TRIPLES = [(i, j, k) for i in range(3) for j in range(i + 1, 4)
            for k in range(j + 1, 5)]


def _tc_body(g_ref, w1t_ref, w2t_ref, wot_ref, attv_ref,
             out_ref, score_ref):
    _, kp, t, c = g_ref.shape
    gi = g_ref[0]
    gj = g_ref[1]
    w1t = w1t_ref[...]
    w2t = w2t_ref[...]
    wot = wot_ref[...]
    attv = attv_ref[...]

    def mm(a3, w):
        a2 = a3.reshape(kp * t, c)
        if a2.dtype != jnp.bfloat16:
            a2 = a2.astype(jnp.bfloat16)
        r = lax.dot_general(a2, w.astype(jnp.bfloat16), (((1,), (0,)), ((), ())),
                            preferred_element_type=jnp.float32,
                            precision=lax.Precision.DEFAULT)
        return r.reshape(kp, t, c)

    a1_0 = mm(gi, w1t)
    a1_1 = mm(gj, w1t)
    a2_0 = mm(gi, w2t)
    a2_1 = mm(gj, w2t)
    a1s = [a1_0, a1_1, a1_0 - a1_1]
    d = gi - gj
    nr = jnp.sqrt(jnp.sum(d * d, axis=2, keepdims=True))
    w2sum = jnp.sum(w2t, axis=0, keepdims=True)[None]
    a2s = [a2_0, a2_1, a2_0 - a2_1,
           a2_0 - jnp.mean(a2_0, axis=0, keepdims=True),
           nr * w2sum]

    firs = {}
    f1s = {}
    for (i, j) in _PAIRS:
        fir = jnp.maximum(a1s[i] + a2s[j], 0.0)
        firs[(i, j)] = fir
        f1s[(i, j)] = mm(fir, w1t)
    nn = firs[(0, 1)]

    nn_hi = jnp.max(nn, axis=0, keepdims=True)
    nn_lo = jnp.min(nn, axis=0, keepdims=True)
    contribs = []
    for o, (i, j, k) in enumerate(_TRIPLES):
        op = jnp.maximum(f1s[(i, j)] + a2s[k], 0.0)
        out_ref[o] = jnp.maximum(jnp.max(mm(op, wot), axis=0), 0.0)
        cv = attv[o][None, None, :]
        base = jnp.where(cv >= 0, nn_hi, nn_lo)
        e = jnp.exp(cv * (nn - base))
        s = jnp.sum(e, axis=0)
        w_ = jnp.sum(e * op, axis=0)
        contribs.append(jnp.sum(w_ / s, axis=0)[None, :])
    contribs = jnp.concatenate(
        contribs + [jnp.zeros((6, c), jnp.float32)], axis=0)

    @pl.when(pl.program_id(0) == 0)
    def _():
        score_ref[...] = contribs

    @pl.when(pl.program_id(0) != 0)
    def _():
        score_ref[...] = score_ref[...] + contribs


def _tc_call(g4, w1t, w2t, wot, attv):
    _, kp, n, c = g4.shape
    t_ = 200
    return pl.pallas_call(
        _tc_body,
        grid=(n // t_,),
        in_specs=[
            pl.BlockSpec((2, kp, t_, c), lambda i: (0, 0, i, 0)),
            pl.BlockSpec((c, c), lambda i: (0, 0)),
            pl.BlockSpec((c, c), lambda i: (0, 0)),
            pl.BlockSpec((c, c), lambda i: (0, 0)),
            pl.BlockSpec((16, c), lambda i: (0, 0)),
        ],
        out_specs=[
            pl.BlockSpec((10, t_, c), lambda i: (0, i, 0)),
            pl.BlockSpec((16, c), lambda i: (0, 0)),
        ],
        out_shape=[
            jax.ShapeDtypeStruct((10, n, c), jnp.float32),
            jax.ShapeDtypeStruct((16, c), jnp.float32),
        ],
    )(g4, w1t, w2t, wot, attv)


def kernel(x, edge_index, W_nn, W_out, att):
    n = x.shape[2]
    c = x.shape[1]
    k = edge_index.shape[3]
    kp = k + 1
    xT = jnp.transpose(x[0, :, :, 0])
    ei = edge_index.astype(jnp.int32)
    self_row = jnp.arange(n, dtype=jnp.int32)[None]
    idx_i = jnp.concatenate([jnp.transpose(ei[1, 0]), self_row], axis=0)
    idx_j = jnp.concatenate([jnp.transpose(ei[0, 0]), self_row], axis=0)

    n2 = n // 2
    r_h = 2 * kp * n2
    r_pad = ((r_h + _NW * _CHUNK - 1) // (_NW * _CHUNK)) * (_NW * _CHUNK)
    g4s = []
    for h in range(2):
        sl = slice(h * n2, (h + 1) * n2)
        idx_h = jnp.concatenate(
            [idx_i[:, sl].reshape(-1), idx_j[:, sl].reshape(-1),
             jnp.zeros((r_pad - r_h,), jnp.int32)])
        g = _sc_gather(xT, idx_h, r_pad)
        g4s.append(g[:r_h].reshape(2, kp, n2, c))

    w1t = jnp.transpose(W_nn[:, :c])
    w2t = jnp.transpose(W_nn[:, c:])
    wot = jnp.transpose(W_out)
    cv = jnp.where(att >= 0, att, 0.2 * att).reshape(10, 1)
    attv = jnp.zeros((16, c), jnp.float32).at[:10].set(
        jnp.broadcast_to(cv, (10, c)))

    out0, part0 = _tc_call(g4s[0], w1t, w2t, wot, attv)
    out1, part1 = _tc_call(g4s[1], w1t, w2t, wot, attv)

    score = jnp.sum(part0[:10] + part1[:10], axis=1)
    index = jnp.argmax(score)
    sel = jnp.concatenate(
        [jnp.take(out0, index, axis=0), jnp.take(out1, index, axis=0)],
        axis=0)
    return jnp.transpose(sel)[None, :, :, None]
